# baseline (speedup 1.0000x reference)
import functools

import jax
import jax.numpy as jnp
from jax import lax
from jax.experimental import pallas as pl
from jax.experimental.pallas import tpu as pltpu
from jax.experimental.pallas import tpu_sc as plsc

D_MODEL = 1024
MAX_REL = 10
N_TABLE = 2 * MAX_REL + 1

_SC_INFO = plsc.get_sparse_core_info()
_NC = _SC_INFO.num_cores
_NS = _SC_INFO.num_subcores
_NW = _NC * _NS
_LANES = _SC_INFO.num_lanes

CHUNK = 32
N_ROWS = 2 * 4096 - 1
N_SLOTS = (N_ROWS + CHUNK - 1) // CHUNK
N_FULL = N_ROWS // CHUNK
SLOTS_PER_W = N_SLOTS // _NW
SPAN = CHUNK * SLOTS_PER_W
TAIL_START = N_FULL * CHUNK
SHIFT = N_ROWS // 2 - MAX_REL
BAND_LO = SHIFT + 1
BAND_HI = SHIFT + N_TABLE - 2


def _emb_sc(pe):
    mesh = plsc.VectorSubcoreMesh(core_axis_name="c", subcore_axis_name="s")

    @functools.partial(
        pl.kernel,
        mesh=mesh,
        out_type=jax.ShapeDtypeStruct((N_ROWS, D_MODEL), jnp.float32),
        scratch_types=[
            pltpu.VMEM((CHUNK,), jnp.int32),
            pltpu.VMEM((CHUNK,), jnp.int32),
            pltpu.VMEM((CHUNK, D_MODEL), jnp.float32),
            pltpu.VMEM((CHUNK, D_MODEL), jnp.float32),
            pltpu.SemaphoreType.DMA,
            pltpu.SemaphoreType.DMA,
            pltpu.SemaphoreType.DMA,
        ],
    )
    def k(pe_hbm, out_hbm, gidx, tidx, buf_const, buf_mix, gsem, csem, msem):
        wid = lax.axis_index("s") * _NC + lax.axis_index("c")
        iota = lax.iota(jnp.int32, _LANES)
        span_start = wid * SPAN

        below = span_start < BAND_LO
        t_const = jnp.clip(
            jnp.where(below, span_start, span_start + SPAN - 1) - SHIFT,
            0, N_TABLE - 1)
        for g in range(CHUNK // _LANES):
            gidx[pl.ds(g * _LANES, _LANES)] = iota * 0 + t_const
        pltpu.async_copy(pe_hbm.at[gidx], buf_const, gsem).wait()

        mixed_any = (span_start <= BAND_HI) & (span_start + SPAN - 1 >= BAND_LO)
        c_mix = jnp.where(below, BAND_LO // CHUNK, BAND_HI // CHUNK)

        @pl.when(mixed_any)
        def _fill_mix():
            s = c_mix * CHUNK
            for g in range(CHUNK // _LANES):
                gidx[pl.ds(g * _LANES, _LANES)] = jnp.clip(
                    s + g * _LANES + iota - SHIFT, 0, N_TABLE - 1)
            pltpu.async_copy(pe_hbm.at[gidx], buf_mix, gsem).wait()

        descs = []
        for k_step in range(SLOTS_PER_W):
            c = wid * SLOTS_PER_W + k_step
            s = pl.multiple_of(c * CHUNK, CHUNK)
            is_mix = mixed_any & (c == c_mix)
            live = c < N_FULL
            dc = pltpu.make_async_copy(
                buf_const, out_hbm.at[pl.ds(s, CHUNK)], csem)
            dm = pltpu.make_async_copy(
                buf_mix, out_hbm.at[pl.ds(s, CHUNK)], msem)

            @pl.when(live & jnp.logical_not(is_mix))
            def _(dc=dc):
                dc.start()

            @pl.when(live & is_mix)
            def _(dm=dm):
                dm.start()
            descs.append((live, is_mix, dc, dm))

        for live, is_mix, dc, dm in descs:
            @pl.when(live & jnp.logical_not(is_mix))
            def _(dc=dc):
                dc.wait()

            @pl.when(live & is_mix)
            def _(dm=dm):
                dm.wait()

        @pl.when(wid == _NW - 1)
        def _tail():
            for g in range(CHUNK // _LANES):
                r = jnp.minimum(TAIL_START + g * _LANES + iota, N_ROWS - 1)
                gidx[pl.ds(g * _LANES, _LANES)] = jnp.clip(
                    r - SHIFT, 0, N_TABLE - 1)
                tidx[pl.ds(g * _LANES, _LANES)] = r
            pltpu.async_copy(pe_hbm.at[gidx], buf_mix, gsem).wait()
            pltpu.async_copy(buf_mix, out_hbm.at[tidx], msem).wait()

    return k(pe)


_COPY_CHUNKS = 8


def _copy_body(x_ref, o_ref, *sems):
    rows = x_ref.shape[0] // _COPY_CHUNKS
    descs = [
        pltpu.make_async_copy(
            x_ref.at[pl.ds(i * rows, rows)],
            o_ref.at[pl.ds(i * rows, rows)], sems[i])
        for i in range(_COPY_CHUNKS)
    ]
    for d in descs:
        d.start()
    for d in descs:
        d.wait()


def _passthrough_tc(x):
    b, t, d = x.shape
    flat = x.reshape(b * t, d)
    out = pl.pallas_call(
        _copy_body,
        in_specs=[pl.BlockSpec(memory_space=pl.ANY)],
        out_specs=pl.BlockSpec(memory_space=pl.ANY),
        out_shape=jax.ShapeDtypeStruct(flat.shape, x.dtype),
        scratch_shapes=[pltpu.SemaphoreType.DMA] * _COPY_CHUNKS,
    )(flat)
    return out.reshape(b, t, d)


def kernel(x, pe):
    emb = _emb_sc(pe)
    x_out = _passthrough_tc(x)
    return (x_out, emb)


# speedup vs baseline: 17.9169x; 17.9169x over previous
import functools

import jax
import jax.numpy as jnp
from jax import lax
from jax.experimental import pallas as pl
from jax.experimental.pallas import tpu as pltpu
from jax.experimental.pallas import tpu_sc as plsc

D_MODEL = 1024
MAX_REL = 10
N_TABLE = 2 * MAX_REL + 1

_SC_INFO = plsc.get_sparse_core_info()
_NC = _SC_INFO.num_cores
_NS = _SC_INFO.num_subcores
_NW = _NC * _NS
_LANES = _SC_INFO.num_lanes

CHUNK = 32
N_ROWS = 2 * 4096 - 1
N_SLOTS = (N_ROWS + CHUNK - 1) // CHUNK
N_FULL = N_ROWS // CHUNK
SLOTS_PER_W = N_SLOTS // _NW
SPAN = CHUNK * SLOTS_PER_W
TAIL_START = N_FULL * CHUNK
SHIFT = N_ROWS // 2 - MAX_REL
BAND_LO = SHIFT + 1
BAND_HI = SHIFT + N_TABLE - 2


def _emb_sc(pe):
    mesh = plsc.VectorSubcoreMesh(core_axis_name="c", subcore_axis_name="s")

    @functools.partial(
        pl.kernel,
        mesh=mesh,
        out_type=jax.ShapeDtypeStruct((N_ROWS, D_MODEL), jnp.float32),
        scratch_types=[
            pltpu.VMEM((CHUNK,), jnp.int32),
            pltpu.VMEM((CHUNK,), jnp.int32),
            pltpu.VMEM((CHUNK, D_MODEL), jnp.float32),
            pltpu.VMEM((CHUNK, D_MODEL), jnp.float32),
            pltpu.SemaphoreType.DMA,
            pltpu.SemaphoreType.DMA,
            pltpu.SemaphoreType.DMA,
        ],
    )
    def k(pe_hbm, out_hbm, gidx, tidx, buf_const, buf_mix, gsem, csem, msem):
        wid = lax.axis_index("s") * _NC + lax.axis_index("c")
        iota = lax.iota(jnp.int32, _LANES)
        span_start = wid * SPAN

        below = span_start < BAND_LO
        t_const = jnp.clip(
            jnp.where(below, span_start, span_start + SPAN - 1) - SHIFT,
            0, N_TABLE - 1)
        for g in range(CHUNK // _LANES):
            gidx[pl.ds(g * _LANES, _LANES)] = iota * 0 + t_const
        pltpu.async_copy(pe_hbm.at[gidx], buf_const, gsem).wait()

        mixed_any = (span_start <= BAND_HI) & (span_start + SPAN - 1 >= BAND_LO)
        c_mix = jnp.where(below, BAND_LO // CHUNK, BAND_HI // CHUNK)

        @pl.when(mixed_any)
        def _fill_mix():
            s = c_mix * CHUNK
            for g in range(CHUNK // _LANES):
                gidx[pl.ds(g * _LANES, _LANES)] = jnp.clip(
                    s + g * _LANES + iota - SHIFT, 0, N_TABLE - 1)
            pltpu.async_copy(pe_hbm.at[gidx], buf_mix, gsem).wait()

        descs = []
        for k_step in range(SLOTS_PER_W):
            c = wid * SLOTS_PER_W + k_step
            s = pl.multiple_of(c * CHUNK, CHUNK)
            is_mix = mixed_any & (c == c_mix)
            live = c < N_FULL
            dc = pltpu.make_async_copy(
                buf_const, out_hbm.at[pl.ds(s, CHUNK)], csem)
            dm = pltpu.make_async_copy(
                buf_mix, out_hbm.at[pl.ds(s, CHUNK)], msem)

            @pl.when(live & jnp.logical_not(is_mix))
            def _(dc=dc):
                dc.start()

            @pl.when(live & is_mix)
            def _(dm=dm):
                dm.start()
            descs.append((live, is_mix, dc, dm))

        for live, is_mix, dc, dm in descs:
            @pl.when(live & jnp.logical_not(is_mix))
            def _(dc=dc):
                dc.wait()

            @pl.when(live & is_mix)
            def _(dm=dm):
                dm.wait()

        @pl.when(wid == _NW - 1)
        def _tail():
            for g in range(CHUNK // _LANES):
                r = jnp.minimum(TAIL_START + g * _LANES + iota, N_ROWS - 1)
                gidx[pl.ds(g * _LANES, _LANES)] = jnp.clip(
                    r - SHIFT, 0, N_TABLE - 1)
                tidx[pl.ds(g * _LANES, _LANES)] = r
            pltpu.async_copy(pe_hbm.at[gidx], buf_mix, gsem).wait()
            pltpu.async_copy(buf_mix, out_hbm.at[tidx], msem).wait()

    return k(pe)


def kernel(x, pe):
    emb = _emb_sc(pe)
    return (x, emb)


# speedup vs baseline: 23.9491x; 1.3367x over previous
import functools

import jax
import jax.numpy as jnp
from jax import lax
from jax.experimental import pallas as pl
from jax.experimental.pallas import tpu as pltpu
from jax.experimental.pallas import tpu_sc as plsc

D_MODEL = 1024
MAX_REL = 10
N_TABLE = 2 * MAX_REL + 1

_SC_INFO = plsc.get_sparse_core_info()
_NC = _SC_INFO.num_cores
_NS = _SC_INFO.num_subcores
_NW = _NC * _NS
_LANES = _SC_INFO.num_lanes

CHUNK = 32
N_ROWS = 2 * 4096 - 1
N_SLOTS = (N_ROWS + CHUNK - 1) // CHUNK
N_FULL = N_ROWS // CHUNK
SLOTS_PER_W = N_SLOTS // _NW
SPAN = CHUNK * SLOTS_PER_W
TAIL_START = N_FULL * CHUNK
SHIFT = N_ROWS // 2 - MAX_REL
BAND_LO = SHIFT + 1
BAND_HI = SHIFT + N_TABLE - 2
LANE_GROUPS = D_MODEL // _LANES


def _emb_sc(pe_flat):
    mesh = plsc.VectorSubcoreMesh(core_axis_name="c", subcore_axis_name="s")

    @functools.partial(
        pl.kernel,
        mesh=mesh,
        out_type=jax.ShapeDtypeStruct((N_ROWS, D_MODEL), jnp.float32),
        scratch_types=[
            pltpu.VMEM((N_TABLE * D_MODEL,), jnp.float32),
            pltpu.VMEM((CHUNK,), jnp.int32),
            pltpu.VMEM((CHUNK, D_MODEL), jnp.float32),
            pltpu.VMEM((CHUNK, D_MODEL), jnp.float32),
            pltpu.SemaphoreType.DMA,
            pltpu.SemaphoreType.DMA,
        ],
    )
    def k(pe_hbm, out_hbm, tab, tidx, buf_const, buf_mix, csem, msem):
        wid = lax.axis_index("s") * _NC + lax.axis_index("c")
        iota = lax.iota(jnp.int32, _LANES)
        span_start = wid * SPAN

        pltpu.sync_copy(pe_hbm, tab)

        below = span_start < BAND_LO
        t_const = jnp.clip(
            jnp.where(below, span_start, span_start + SPAN - 1) - SHIFT,
            0, N_TABLE - 1)

        def fill_const(c, carry):
            v = tab[pl.ds(t_const * D_MODEL + c * _LANES, _LANES)]
            for j in range(CHUNK):
                buf_const[j, pl.ds(c * _LANES, _LANES)] = v
            return carry
        lax.fori_loop(0, LANE_GROUPS, fill_const, 0)

        mixed_any = (span_start <= BAND_HI) & (span_start + SPAN - 1 >= BAND_LO)
        c_mix = jnp.where(below, BAND_LO // CHUNK, BAND_HI // CHUNK)

        @pl.when(mixed_any)
        def _fill_mix():
            s_mix = c_mix * CHUNK
            for j in range(CHUNK):
                t_j = jnp.clip(s_mix + j - SHIFT, 0, N_TABLE - 1)

                def cp(c, carry, t_j=t_j, j=j):
                    buf_mix[j, pl.ds(c * _LANES, _LANES)] = tab[
                        pl.ds(t_j * D_MODEL + c * _LANES, _LANES)]
                    return carry
                lax.fori_loop(0, LANE_GROUPS, cp, 0)

        descs = []
        for k_step in range(SLOTS_PER_W):
            c = wid * SLOTS_PER_W + k_step
            s = pl.multiple_of(c * CHUNK, CHUNK)
            is_mix = mixed_any & (c == c_mix)
            live = c < N_FULL
            dc = pltpu.make_async_copy(
                buf_const, out_hbm.at[pl.ds(s, CHUNK)], csem)
            dm = pltpu.make_async_copy(
                buf_mix, out_hbm.at[pl.ds(s, CHUNK)], msem)

            @pl.when(live & jnp.logical_not(is_mix))
            def _(dc=dc):
                dc.start()

            @pl.when(live & is_mix)
            def _(dm=dm):
                dm.start()
            descs.append((live, is_mix, dc, dm))

        for live, is_mix, dc, dm in descs:
            @pl.when(live & jnp.logical_not(is_mix))
            def _(dc=dc):
                dc.wait()

            @pl.when(live & is_mix)
            def _(dm=dm):
                dm.wait()

        @pl.when(wid == _NW - 1)
        def _tail():
            for g in range(CHUNK // _LANES):
                tidx[pl.ds(g * _LANES, _LANES)] = jnp.minimum(
                    TAIL_START + g * _LANES + iota, N_ROWS - 1)
            pltpu.async_copy(buf_const, out_hbm.at[tidx], msem).wait()

    return k(pe_flat)


def kernel(x, pe):
    emb = _emb_sc(pe.reshape(-1))
    return (x, emb)


# speedup vs baseline: 27.4551x; 1.1464x over previous
import functools

import jax
import jax.numpy as jnp
from jax import lax
from jax.experimental import pallas as pl
from jax.experimental.pallas import tpu as pltpu
from jax.experimental.pallas import tpu_sc as plsc

D_MODEL = 1024
MAX_REL = 10
N_TABLE = 2 * MAX_REL + 1

_SC_INFO = plsc.get_sparse_core_info()
_NC = _SC_INFO.num_cores
_NS = _SC_INFO.num_subcores
_NW = _NC * _NS
_LANES = _SC_INFO.num_lanes

CHUNK = 32
N_ROWS = 2 * 4096 - 1
N_SLOTS = (N_ROWS + CHUNK - 1) // CHUNK
N_FULL = N_ROWS // CHUNK
SLOTS_PER_W = N_SLOTS // _NW
SPAN = CHUNK * SLOTS_PER_W
TAIL_START = N_FULL * CHUNK
SHIFT = N_ROWS // 2 - MAX_REL
BAND_LO = SHIFT + 1
BAND_HI = SHIFT + N_TABLE - 2
LANE_GROUPS = D_MODEL // _LANES


def _emb_sc(pe_flat):
    mesh = plsc.VectorSubcoreMesh(core_axis_name="c", subcore_axis_name="s")

    @functools.partial(
        pl.kernel,
        mesh=mesh,
        out_type=jax.ShapeDtypeStruct((N_ROWS, D_MODEL), jnp.float32),
        scratch_types=[
            pltpu.VMEM((N_TABLE * D_MODEL,), jnp.float32),
            pltpu.VMEM((CHUNK,), jnp.int32),
            pltpu.VMEM((CHUNK, D_MODEL), jnp.float32),
            pltpu.VMEM((CHUNK, D_MODEL), jnp.float32),
            pltpu.SemaphoreType.DMA,
            pltpu.SemaphoreType.DMA,
        ],
    )
    def k(pe_hbm, out_hbm, tab, tidx, buf_const, buf_mix, csem, msem):
        wid = lax.axis_index("s") * _NC + lax.axis_index("c")
        iota = lax.iota(jnp.int32, _LANES)
        span_start = wid * SPAN

        pltpu.sync_copy(pe_hbm, tab)

        below = span_start < BAND_LO
        t_const = jnp.clip(
            jnp.where(below, span_start, span_start + SPAN - 1) - SHIFT,
            0, N_TABLE - 1)

        def fill_const(c, carry):
            v = tab[pl.ds(t_const * D_MODEL + c * _LANES, _LANES)]
            for j in range(CHUNK):
                buf_const[j, pl.ds(c * _LANES, _LANES)] = v
            return carry
        lax.fori_loop(0, LANE_GROUPS, fill_const, 0)

        mixed_any = (span_start <= BAND_HI) & (span_start + SPAN - 1 >= BAND_LO)
        c_mix = jnp.where(below, BAND_LO // CHUNK, BAND_HI // CHUNK)

        @pl.when(mixed_any)
        def _fill_mix():
            s_mix = c_mix * CHUNK
            for j in range(CHUNK):
                t_j = jnp.clip(s_mix + j - SHIFT, 0, N_TABLE - 1)

                def cp(c, carry, t_j=t_j, j=j):
                    buf_mix[j, pl.ds(c * _LANES, _LANES)] = tab[
                        pl.ds(t_j * D_MODEL + c * _LANES, _LANES)]
                    return carry
                lax.fori_loop(0, LANE_GROUPS, cp, 0)

        descs = []
        for k_step in range(SLOTS_PER_W):
            c = wid * SLOTS_PER_W + k_step
            s = pl.multiple_of(c * CHUNK, CHUNK)
            is_mix = mixed_any & (c == c_mix)
            live = c < N_FULL
            dc = pltpu.make_async_copy(
                buf_const, out_hbm.at[pl.ds(s, CHUNK)], csem)
            dm = pltpu.make_async_copy(
                buf_mix, out_hbm.at[pl.ds(s, CHUNK)], msem)

            @pl.when(live & jnp.logical_not(is_mix))
            def _(dc=dc):
                dc.start()

            @pl.when(live & is_mix)
            def _(dm=dm):
                dm.start()
            descs.append((live, is_mix, dc, dm))

        for live, is_mix, dc, dm in descs:
            @pl.when(live & jnp.logical_not(is_mix))
            def _(dc=dc):
                dc.wait()

            @pl.when(live & is_mix)
            def _(dm=dm):
                dm.wait()

        @pl.when(wid == _NW - 1)
        def _tail():
            for g in range(CHUNK // _LANES):
                tidx[pl.ds(g * _LANES, _LANES)] = jnp.minimum(
                    TAIL_START + g * _LANES + iota, N_ROWS - 1)
            pltpu.async_copy(buf_const, out_hbm.at[tidx], msem).wait()

    return k(pe_flat)


_COPY_ROWS = 1024


def _copy_block(i_ref, o_ref):
    o_ref[...] = i_ref[...]


def _passthrough_tc(x):
    b, t, d = x.shape
    flat = x.reshape(b * t, d)
    out = pl.pallas_call(
        _copy_block,
        grid=(flat.shape[0] // _COPY_ROWS,),
        in_specs=[pl.BlockSpec((_COPY_ROWS, d), lambda i: (i, 0))],
        out_specs=pl.BlockSpec((_COPY_ROWS, d), lambda i: (i, 0)),
        out_shape=jax.ShapeDtypeStruct(flat.shape, x.dtype),
    )(flat)
    return out.reshape(b, t, d)


def kernel(x, pe):
    emb = _emb_sc(pe.reshape(-1))
    return (_passthrough_tc(x), emb)


# speedup vs baseline: 28.6009x; 1.0417x over previous
import functools

import jax
import jax.numpy as jnp
from jax import lax
from jax.experimental import pallas as pl
from jax.experimental.pallas import tpu as pltpu
from jax.experimental.pallas import tpu_sc as plsc

D_MODEL = 1024
MAX_REL = 10
N_TABLE = 2 * MAX_REL + 1

_SC_INFO = plsc.get_sparse_core_info()
_NC = _SC_INFO.num_cores
_NS = _SC_INFO.num_subcores
_NW = _NC * _NS
_LANES = _SC_INFO.num_lanes

CHUNK = 32
N_ROWS = 2 * 4096 - 1
N_SLOTS = (N_ROWS + CHUNK - 1) // CHUNK
N_FULL = N_ROWS // CHUNK
SLOTS_PER_W = N_SLOTS // _NW
SPAN = CHUNK * SLOTS_PER_W
TAIL_START = N_FULL * CHUNK
SHIFT = N_ROWS // 2 - MAX_REL
BAND_LO = SHIFT + 1
BAND_HI = SHIFT + N_TABLE - 2
LANE_GROUPS = D_MODEL // _LANES


def _emb_sc(pe_flat):
    mesh = plsc.VectorSubcoreMesh(core_axis_name="c", subcore_axis_name="s")

    @functools.partial(
        pl.kernel,
        mesh=mesh,
        out_type=jax.ShapeDtypeStruct((N_ROWS, D_MODEL), jnp.float32),
        scratch_types=[
            pltpu.VMEM((N_TABLE * D_MODEL,), jnp.float32),
            pltpu.VMEM((CHUNK,), jnp.int32),
            pltpu.VMEM((CHUNK, D_MODEL), jnp.float32),
            pltpu.VMEM((CHUNK, D_MODEL), jnp.float32),
            pltpu.SemaphoreType.DMA,
            pltpu.SemaphoreType.DMA,
        ],
    )
    def k(pe_hbm, out_hbm, tab, tidx, buf_const, buf_mix, csem, msem):
        wid = lax.axis_index("s") * _NC + lax.axis_index("c")
        iota = lax.iota(jnp.int32, _LANES)
        span_start = wid * SPAN

        pltpu.sync_copy(pe_hbm, tab)

        below = span_start < BAND_LO
        t_const = jnp.clip(
            jnp.where(below, span_start, span_start + SPAN - 1) - SHIFT,
            0, N_TABLE - 1)

        def fill_const(c, carry):
            v = tab[pl.ds(t_const * D_MODEL + c * _LANES, _LANES)]
            for j in range(CHUNK):
                buf_const[j, pl.ds(c * _LANES, _LANES)] = v
            return carry
        lax.fori_loop(0, LANE_GROUPS, fill_const, 0)

        mixed_any = (span_start <= BAND_HI) & (span_start + SPAN - 1 >= BAND_LO)
        c_mix = jnp.where(below, BAND_LO // CHUNK, BAND_HI // CHUNK)

        @pl.when(mixed_any)
        def _fill_mix():
            s_mix = c_mix * CHUNK
            for j in range(CHUNK):
                t_j = jnp.clip(s_mix + j - SHIFT, 0, N_TABLE - 1)

                def cp(c, carry, t_j=t_j, j=j):
                    buf_mix[j, pl.ds(c * _LANES, _LANES)] = tab[
                        pl.ds(t_j * D_MODEL + c * _LANES, _LANES)]
                    return carry
                lax.fori_loop(0, LANE_GROUPS, cp, 0)

        descs = []
        for k_step in range(SLOTS_PER_W):
            c = wid * SLOTS_PER_W + k_step
            s = pl.multiple_of(c * CHUNK, CHUNK)
            is_mix = mixed_any & (c == c_mix)
            live = c < N_FULL
            dc = pltpu.make_async_copy(
                buf_const, out_hbm.at[pl.ds(s, CHUNK)], csem)
            dm = pltpu.make_async_copy(
                buf_mix, out_hbm.at[pl.ds(s, CHUNK)], msem)

            @pl.when(live & jnp.logical_not(is_mix))
            def _(dc=dc):
                dc.start()

            @pl.when(live & is_mix)
            def _(dm=dm):
                dm.start()
            descs.append((live, is_mix, dc, dm))

        for live, is_mix, dc, dm in descs:
            @pl.when(live & jnp.logical_not(is_mix))
            def _(dc=dc):
                dc.wait()

            @pl.when(live & is_mix)
            def _(dm=dm):
                dm.wait()

        @pl.when(wid == _NW - 1)
        def _tail():
            for g in range(CHUNK // _LANES):
                tidx[pl.ds(g * _LANES, _LANES)] = jnp.minimum(
                    TAIL_START + g * _LANES + iota, N_ROWS - 1)
            pltpu.async_copy(buf_const, out_hbm.at[tidx], msem).wait()

    return k(pe_flat)


_COPY_ROWS = 2048


def _copy_block(i_ref, o_ref):
    o_ref[...] = i_ref[...]


def _passthrough_tc(x):
    b, t, d = x.shape
    flat = x.reshape(b * t, d)
    out = pl.pallas_call(
        _copy_block,
        grid=(flat.shape[0] // _COPY_ROWS,),
        in_specs=[pl.BlockSpec((_COPY_ROWS, d), lambda i: (i, 0))],
        out_specs=pl.BlockSpec((_COPY_ROWS, d), lambda i: (i, 0)),
        out_shape=jax.ShapeDtypeStruct(flat.shape, x.dtype),
    )(flat)
    return out.reshape(b, t, d)


def kernel(x, pe):
    emb = _emb_sc(pe.reshape(-1))
    return (_passthrough_tc(x), emb)
